# Initial kernel scaffold; baseline (speedup 1.0000x reference)
import functools

import jax
import jax.numpy as jnp
from jax import lax
from jax.experimental import pallas as pl
from jax.experimental.pallas import tpu as pltpu
from jax.experimental.pallas import tpu_sc as plsc

N = 10000
E = 160000
D_IN = 256
D_H = 720
D_OUT = 360
G = 8
EPS = 1e-5

NC = 2
NS = 16
NW = NC * NS
EB = 128
NB = 40
EW = EB * NB
E_PAD = EW * NW
N_ACC = 10240
RPT = N_ACC // NS

BR = 400
NBLK = N // BR

_mesh = plsc.VectorSubcoreMesh(core_axis_name="c", subcore_axis_name="s")


@functools.partial(
    pl.kernel,
    out_type=jax.ShapeDtypeStruct((2 * N_ACC, 16), jnp.float32),
    mesh=_mesh,
    scratch_types=[
        pltpu.VMEM((EB,), jnp.int32),
        pltpu.VMEM((EB, 16), jnp.float32),
        pltpu.VMEM((64, 16), jnp.float32),
        pltpu.VMEM_SHARED((N_ACC, 16), jnp.float32),
    ],
)
def _sc_degree(dst_hbm, out_hbm, idx_d, ones, zbuf, acc):
    c = lax.axis_index("c")
    s = lax.axis_index("s")
    w = s * NC + c
    o16 = jnp.ones((16,), jnp.float32)
    z16 = jnp.zeros((16,), jnp.float32)

    @pl.loop(0, EB)
    def _fill_ones(i):
        ones[i, :] = o16

    @pl.loop(0, 64)
    def _fill_zero(i):
        zbuf[i, :] = z16

    @pl.loop(0, RPT // 64)
    def _zero_acc(k):
        pltpu.sync_copy(zbuf, acc.at[pl.ds(s * RPT + k * 64, 64)])

    plsc.subcore_barrier()

    @pl.loop(0, NB)
    def _edges(b):
        pltpu.sync_copy(dst_hbm.at[pl.ds(w * EW + b * EB, EB)], idx_d)
        pltpu.sync_copy(ones, acc.at[idx_d], add=True)

    plsc.subcore_barrier()
    pltpu.sync_copy(acc.at[pl.ds(s * RPT, RPT)],
                    out_hbm.at[pl.ds(c * N_ACC + s * RPT, RPT)])


def _make_propagate(dh):
    @functools.partial(
        pl.kernel,
        out_type=jax.ShapeDtypeStruct((2 * N_ACC, dh), jnp.float32),
        mesh=_mesh,
        scratch_types=[
            pltpu.VMEM((EB,), jnp.int32),
            pltpu.VMEM((EB,), jnp.int32),
            pltpu.VMEM((EB, dh), jnp.float32),
            pltpu.VMEM((64, dh), jnp.float32),
            pltpu.VMEM_SHARED((N_ACC, dh), jnp.float32),
            pltpu.SemaphoreType.DMA,
        ],
    )
    def _prop(f_hbm, src_hbm, dst_hbm, out_hbm, idx_s, idx_d, rows, zbuf, acc,
              sem):
        c = lax.axis_index("c")
        s = lax.axis_index("s")
        w = s * NC + c
        coff = c * N
        z16 = jnp.zeros((16,), jnp.float32)

        @pl.loop(0, 64)
        def _fill_zero(i):
            for j in range(dh // 16):
                zbuf[i, pl.ds(j * 16, 16)] = z16

        @pl.loop(0, RPT // 64)
        def _zero_acc(k):
            pltpu.sync_copy(zbuf, acc.at[pl.ds(s * RPT + k * 64, 64)])

        plsc.subcore_barrier()

        @pl.loop(0, NB)
        def _edges(b):
            off = w * EW + b * EB
            pltpu.sync_copy(src_hbm.at[pl.ds(off, EB)], idx_s)
            pltpu.sync_copy(dst_hbm.at[pl.ds(off, EB)], idx_d)
            for j in range(EB // 16):
                sl = pl.ds(j * 16, 16)
                idx_s[sl] = idx_s[sl] + coff
            pltpu.async_copy(f_hbm.at[idx_s], rows, sem).wait()
            pltpu.sync_copy(rows, acc.at[idx_d], add=True)

        plsc.subcore_barrier()
        pltpu.sync_copy(acc.at[pl.ds(s * RPT, RPT)],
                        out_hbm.at[pl.ds(c * N_ACC + s * RPT, RPT)])

    return _prop


_sc_prop128 = _make_propagate(128)
_sc_prop192 = _make_propagate(192)


def _mish(x):
    return x * jnp.tanh(jax.nn.softplus(x))


def _tc_prep(x, degf):
    def body(x_ref, d_ref, f_ref, dis_ref):
        deg = d_ref[0, :, 0:1] + d_ref[1, :, 0:1]
        dis = jnp.where(deg > 0, lax.rsqrt(deg), 0.0)
        f = x_ref[...] * dis
        f_ref[0] = f[:, :128]
        f_ref[1] = f[:, 128:]
        dis_ref[...] = dis

    return pl.pallas_call(
        body,
        grid=(NBLK,),
        in_specs=[
            pl.BlockSpec((BR, D_IN), lambda i: (i, 0)),
            pl.BlockSpec((2, BR, 16), lambda i: (0, i, 0)),
        ],
        out_specs=[
            pl.BlockSpec((2, BR, 128), lambda i: (0, i, 0)),
            pl.BlockSpec((BR, 1), lambda i: (i, 0)),
        ],
        out_shape=[
            jax.ShapeDtypeStruct((2, N, 128), jnp.float32),
            jax.ShapeDtypeStruct((N, 1), jnp.float32),
        ],
    )(x, degf)


def _tc_mm1(s1, dis, W1, b1):
    def body(s_ref, dis_ref, w_ref, b_ref, o_ref, ps_ref, pq_ref):
        t = jnp.concatenate([s_ref[0], s_ref[1]], axis=1) * dis_ref[...]
        o = jnp.dot(t, w_ref[...], preferred_element_type=jnp.float32)
        o = o + b_ref[...]
        o_ref[...] = o
        ps_ref[...] = jnp.sum(o, axis=0, keepdims=True)
        pq_ref[...] = jnp.sum(o * o, axis=0, keepdims=True)

    return pl.pallas_call(
        body,
        grid=(NBLK,),
        in_specs=[
            pl.BlockSpec((2, BR, 128), lambda i: (0, i, 0)),
            pl.BlockSpec((BR, 1), lambda i: (i, 0)),
            pl.BlockSpec((D_IN, D_H), lambda i: (0, 0)),
            pl.BlockSpec((1, D_H), lambda i: (0, 0)),
        ],
        out_specs=[
            pl.BlockSpec((BR, D_H), lambda i: (i, 0)),
            pl.BlockSpec((1, D_H), lambda i: (i, 0)),
            pl.BlockSpec((1, D_H), lambda i: (i, 0)),
        ],
        out_shape=[
            jax.ShapeDtypeStruct((N, D_H), jnp.float32),
            jax.ShapeDtypeStruct((NBLK, D_H), jnp.float32),
            jax.ShapeDtypeStruct((NBLK, D_H), jnp.float32),
        ],
    )(s1, dis, W1, b1)


def _tc_stats(ps, pq, g, be):
    d = ps.shape[1]

    def body(ps_ref, pq_ref, g_ref, be_ref, ac_ref):
        mean = jnp.sum(ps_ref[...], axis=0, keepdims=True) * (1.0 / N)
        ex2 = jnp.sum(pq_ref[...], axis=0, keepdims=True) * (1.0 / N)
        var = ex2 - mean * mean
        a = g_ref[...] * lax.rsqrt(var + EPS)
        c = be_ref[...] - mean * a
        ac_ref[...] = jnp.concatenate([a, c], axis=0)

    return pl.pallas_call(
        body,
        out_shape=jax.ShapeDtypeStruct((2, d), jnp.float32),
    )(ps, pq, g, be)


def _tc_mid(out1, ac1, dis, W2):
    def body(o_ref, ac_ref, dis_ref, w_ref, f_ref):
        h = o_ref[...] * ac_ref[0:1] + ac_ref[1:2]
        h = _mish(h)
        z = jnp.dot(h, w_ref[...], preferred_element_type=jnp.float32)
        zd = z * dis_ref[...]
        zp = jnp.concatenate([zd, jnp.zeros((BR, 24), jnp.float32)], axis=1)
        f_ref[0] = zp[:, :192]
        f_ref[1] = zp[:, 192:]

    return pl.pallas_call(
        body,
        grid=(NBLK,),
        in_specs=[
            pl.BlockSpec((BR, D_H), lambda i: (i, 0)),
            pl.BlockSpec((2, D_H), lambda i: (0, 0)),
            pl.BlockSpec((BR, 1), lambda i: (i, 0)),
            pl.BlockSpec((D_H, D_OUT), lambda i: (0, 0)),
        ],
        out_specs=pl.BlockSpec((2, BR, 192), lambda i: (0, i, 0)),
        out_shape=jax.ShapeDtypeStruct((2, N, 192), jnp.float32),
    )(out1, ac1, dis, W2)


def _tc_recon(s2, dis, b2):
    def body(s_ref, dis_ref, b_ref, o_ref, ps_ref, pq_ref):
        t = jnp.concatenate([s_ref[0], s_ref[1]], axis=1)[:, :D_OUT]
        o = t * dis_ref[...] + b_ref[...]
        o_ref[...] = o
        ps_ref[...] = jnp.sum(o, axis=0, keepdims=True)
        pq_ref[...] = jnp.sum(o * o, axis=0, keepdims=True)

    return pl.pallas_call(
        body,
        grid=(NBLK,),
        in_specs=[
            pl.BlockSpec((2, BR, 192), lambda i: (0, i, 0)),
            pl.BlockSpec((BR, 1), lambda i: (i, 0)),
            pl.BlockSpec((1, D_OUT), lambda i: (0, 0)),
        ],
        out_specs=[
            pl.BlockSpec((BR, D_OUT), lambda i: (i, 0)),
            pl.BlockSpec((1, D_OUT), lambda i: (i, 0)),
            pl.BlockSpec((1, D_OUT), lambda i: (i, 0)),
        ],
        out_shape=[
            jax.ShapeDtypeStruct((N, D_OUT), jnp.float32),
            jax.ShapeDtypeStruct((NBLK, D_OUT), jnp.float32),
            jax.ShapeDtypeStruct((NBLK, D_OUT), jnp.float32),
        ],
    )(s2, dis, b2)


def _tc_pool(out2, ac2, batch):
    def body(o_ref, ac_ref, bt_ref, p_ref):
        i = pl.program_id(0)
        h = o_ref[...] * ac_ref[0:1] + ac_ref[1:2]
        h = _mish(h)
        bt = bt_ref[...]
        rows = [
            jnp.max(jnp.where(bt == g, h, -jnp.inf), axis=0, keepdims=True)
            for g in range(G)
        ]
        bm = jnp.concatenate(rows, axis=0)

        @pl.when(i == 0)
        def _():
            p_ref[...] = bm

        @pl.when(i > 0)
        def _():
            p_ref[...] = jnp.maximum(p_ref[...], bm)

    return pl.pallas_call(
        body,
        grid=(NBLK,),
        in_specs=[
            pl.BlockSpec((BR, D_OUT), lambda i: (i, 0)),
            pl.BlockSpec((2, D_OUT), lambda i: (0, 0)),
            pl.BlockSpec((BR, 1), lambda i: (i, 0)),
        ],
        out_specs=pl.BlockSpec((G, D_OUT), lambda i: (0, 0)),
        out_shape=jax.ShapeDtypeStruct((G, D_OUT), jnp.float32),
    )(out2, ac2, batch)


def _tc_decoder(p1, p2, Wm, bm, Wc, bc):
    def body(p1_ref, p2_ref, wm_ref, bm_ref, wc_ref, bc_ref, o_ref):
        gg = p1_ref[...] * p2_ref[...]
        t = jnp.dot(gg, wm_ref[...], preferred_element_type=jnp.float32)
        t = _mish(t + bm_ref[...])
        o = jnp.dot(t, wc_ref[...], preferred_element_type=jnp.float32)
        o_ref[...] = o + bc_ref[...]

    return pl.pallas_call(
        body,
        out_shape=jax.ShapeDtypeStruct((G, bc.shape[1]), jnp.float32),
    )(p1, p2, Wm, bm, Wc, bc)


def _encode_pool(x, edge_index, batch, W1, b1, g1, be1, W2, b2, g2, be2):
    src = edge_index[0]
    dst = edge_index[1]
    src_p = jnp.concatenate([src, jnp.zeros((E_PAD - E,), jnp.int32)])
    dst_p = jnp.concatenate([dst, jnp.full((E_PAD - E,), N, jnp.int32)])

    degf = _sc_degree(dst_p).reshape(2, N_ACC, 16)
    f1, dis = _tc_prep(x, degf)
    s1 = _sc_prop128(f1.reshape(2 * N, 128), src_p, dst_p)
    s1 = s1.reshape(2, N_ACC, 128)
    out1, ps1, pq1 = _tc_mm1(s1, dis, W1, b1.reshape(1, D_H))
    ac1 = _tc_stats(ps1, pq1, g1.reshape(1, D_H), be1.reshape(1, D_H))
    f2 = _tc_mid(out1, ac1, dis, W2)
    s2 = _sc_prop192(f2.reshape(2 * N, 192), src_p, dst_p)
    s2 = s2.reshape(2, N_ACC, 192)
    out2, ps2, pq2 = _tc_recon(s2, dis, b2.reshape(1, D_OUT))
    ac2 = _tc_stats(ps2, pq2, g2.reshape(1, D_OUT), be2.reshape(1, D_OUT))
    return _tc_pool(out2, ac2, batch.reshape(N, 1))


def kernel(protein1_feat, protein1_edge_index, protein1_nodes_batch,
           protein2_feat, protein2_edge_index, protein2_nodes_batch,
           W1, b1, g1, be1, W2, b2, g2, be2, Wm, bm, Wc, bc):
    p1 = _encode_pool(protein1_feat, protein1_edge_index, protein1_nodes_batch,
                      W1, b1, g1, be1, W2, b2, g2, be2)
    p2 = _encode_pool(protein2_feat, protein2_edge_index, protein2_nodes_batch,
                      W1, b1, g1, be1, W2, b2, g2, be2)
    return _tc_decoder(p1, p2, Wm, bm.reshape(1, -1), Wc,
                       bc.reshape(1, -1))


# speedup vs baseline: 2.3838x; 2.3838x over previous
import functools

import jax
import jax.numpy as jnp
from jax import lax
from jax.experimental import pallas as pl
from jax.experimental.pallas import tpu as pltpu
from jax.experimental.pallas import tpu_sc as plsc

N = 10000
E = 160000
D_IN = 256
D_H = 720
D_OUT = 360
G = 8
EPS = 1e-5

NC = 2
NS = 16
NW = NC * NS
EB = 128
NB = 40
EW = EB * NB
E_PAD = EW * NW
N_ACC = 10240
RPT = N_ACC // NS

BR = 400
NBLK = N // BR

_mesh = plsc.VectorSubcoreMesh(core_axis_name="c", subcore_axis_name="s")


@functools.partial(
    pl.kernel,
    out_type=jax.ShapeDtypeStruct((2 * N_ACC, 128), jnp.float32),
    mesh=_mesh,
    scratch_types=[
        pltpu.VMEM((EB,), jnp.int32),
        pltpu.VMEM((EB, 128), jnp.float32),
        pltpu.VMEM((64, 128), jnp.float32),
        pltpu.VMEM_SHARED((N_ACC, 128), jnp.float32),
    ],
)
def _sc_degree(dst_hbm, out_hbm, idx_d, ones, zbuf, acc):
    c = lax.axis_index("c")
    s = lax.axis_index("s")
    w = s * NC + c
    o16 = jnp.ones((16,), jnp.float32)
    z16 = jnp.zeros((16,), jnp.float32)

    @pl.loop(0, EB)
    def _fill_ones(i):
        for j in range(8):
            ones[i, pl.ds(j * 16, 16)] = o16

    @pl.loop(0, 64)
    def _fill_zero(i):
        for j in range(8):
            zbuf[i, pl.ds(j * 16, 16)] = z16

    @pl.loop(0, RPT // 64)
    def _zero_acc(k):
        pltpu.sync_copy(zbuf, acc.at[pl.ds(s * RPT + k * 64, 64)])

    plsc.subcore_barrier()

    @pl.loop(0, NB)
    def _edges(b):
        pltpu.sync_copy(dst_hbm.at[pl.ds(w * EW + b * EB, EB)], idx_d)
        pltpu.sync_copy(ones, acc.at[idx_d], add=True)

    plsc.subcore_barrier()
    pltpu.sync_copy(acc.at[pl.ds(s * RPT, RPT)],
                    out_hbm.at[pl.ds(c * N_ACC + s * RPT, RPT)])


def _make_propagate(n_chunks):
    rounds = (n_chunks + NC - 1) // NC

    @functools.partial(
        pl.kernel,
        out_type=jax.ShapeDtypeStruct((n_chunks * N_ACC, 128), jnp.float32),
        mesh=_mesh,
        scratch_types=[
            pltpu.VMEM((EB,), jnp.int32),
            pltpu.VMEM((EB,), jnp.int32),
            pltpu.VMEM((EB, 128), jnp.float32),
            pltpu.VMEM((64, 128), jnp.float32),
            pltpu.VMEM_SHARED((N_ACC, 128), jnp.float32),
            pltpu.SemaphoreType.DMA,
        ],
    )
    def _prop(f_hbm, src_hbm, dst_hbm, out_hbm, idx_s, idx_d, rows, zbuf, acc,
              sem):
        c = lax.axis_index("c")
        s = lax.axis_index("s")
        z16 = jnp.zeros((16,), jnp.float32)

        @pl.loop(0, 64)
        def _fill_zero(i):
            for j in range(8):
                zbuf[i, pl.ds(j * 16, 16)] = z16

        for r in range(rounds):
            chunk = r * NC + c

            @pl.when(chunk < n_chunks)
            def _round():
                coff = chunk * N

                @pl.loop(0, RPT // 64)
                def _zero_acc(k):
                    pltpu.sync_copy(zbuf, acc.at[pl.ds(s * RPT + k * 64, 64)])

                plsc.subcore_barrier()

                @pl.loop(0, NB * NC)
                def _edges(b):
                    off = s * (EW * NC) + b * EB
                    pltpu.sync_copy(src_hbm.at[pl.ds(off, EB)], idx_s)
                    pltpu.sync_copy(dst_hbm.at[pl.ds(off, EB)], idx_d)
                    for j in range(EB // 16):
                        sl = pl.ds(j * 16, 16)
                        idx_s[sl] = idx_s[sl] + coff
                    pltpu.async_copy(f_hbm.at[idx_s], rows, sem).wait()
                    pltpu.sync_copy(rows, acc.at[idx_d], add=True)

                plsc.subcore_barrier()
                pltpu.sync_copy(acc.at[pl.ds(s * RPT, RPT)],
                                out_hbm.at[pl.ds(chunk * N_ACC + s * RPT,
                                                 RPT)])

    return _prop


_sc_prop6 = _make_propagate(6)
_sc_prop3 = _make_propagate(3)


def _mish(x):
    return x * jnp.tanh(jax.nn.softplus(x))


def _tc_prep(x, degf, W1):
    def body(x_ref, d_ref, w_ref, f_ref, dis_ref):
        deg = d_ref[0, :, 0:1] + d_ref[1, :, 0:1]
        dis = jnp.where(deg > 0, lax.rsqrt(deg), 0.0)
        h = jnp.dot(x_ref[...], w_ref[...],
                    preferred_element_type=jnp.float32)
        f = h * dis
        zp = jnp.concatenate([f, jnp.zeros((BR, 48), jnp.float32)], axis=1)
        for k in range(6):
            f_ref[k] = zp[:, 128 * k:128 * (k + 1)]
        dis_ref[...] = dis

    return pl.pallas_call(
        body,
        grid=(NBLK,),
        in_specs=[
            pl.BlockSpec((BR, D_IN), lambda i: (i, 0)),
            pl.BlockSpec((2, BR, 128), lambda i: (0, i, 0)),
            pl.BlockSpec((D_IN, D_H), lambda i: (0, 0)),
        ],
        out_specs=[
            pl.BlockSpec((6, BR, 128), lambda i: (0, i, 0)),
            pl.BlockSpec((BR, 1), lambda i: (i, 0)),
        ],
        out_shape=[
            jax.ShapeDtypeStruct((6, N, 128), jnp.float32),
            jax.ShapeDtypeStruct((N, 1), jnp.float32),
        ],
    )(x, degf, W1)


def _tc_stats(ps, pq, g, be):
    d = ps.shape[-1]

    def body(ps_ref, pq_ref, g_ref, be_ref, ac_ref):
        mean = jnp.sum(ps_ref[...], axis=0) * (1.0 / N)
        ex2 = jnp.sum(pq_ref[...], axis=0) * (1.0 / N)
        var = ex2 - mean * mean
        a = g_ref[...] * lax.rsqrt(var + EPS)
        c = be_ref[...] - mean * a
        ac_ref[...] = jnp.concatenate([a, c], axis=0)

    return pl.pallas_call(
        body,
        out_shape=jax.ShapeDtypeStruct((2, d), jnp.float32),
    )(ps, pq, g, be)


def _tc_mid(out1, ac1, dis, W2):
    def body(o_ref, ac_ref, dis_ref, w_ref, f_ref):
        h = o_ref[...] * ac_ref[0:1] + ac_ref[1:2]
        h = _mish(h)
        z = jnp.dot(h, w_ref[...], preferred_element_type=jnp.float32)
        zd = z * dis_ref[...]
        zp = jnp.concatenate([zd, jnp.zeros((BR, 24), jnp.float32)], axis=1)
        f_ref[0] = zp[:, :128]
        f_ref[1] = zp[:, 128:256]
        f_ref[2] = zp[:, 256:]

    return pl.pallas_call(
        body,
        grid=(NBLK,),
        in_specs=[
            pl.BlockSpec((BR, D_H), lambda i: (i, 0)),
            pl.BlockSpec((2, D_H), lambda i: (0, 0)),
            pl.BlockSpec((BR, 1), lambda i: (i, 0)),
            pl.BlockSpec((D_H, D_OUT), lambda i: (0, 0)),
        ],
        out_specs=pl.BlockSpec((3, BR, 128), lambda i: (0, i, 0)),
        out_shape=jax.ShapeDtypeStruct((3, N, 128), jnp.float32),
    )(out1, ac1, dis, W2)


def _tc_recon(s, dis, b, d_out, n_chunks):
    def body(s_ref, dis_ref, b_ref, o_ref, ps_ref, pq_ref):
        t = jnp.concatenate([s_ref[k] for k in range(n_chunks)],
                            axis=1)[:, :d_out]
        o = t * dis_ref[...] + b_ref[...]
        o_ref[...] = o
        ps_ref[...] = jnp.sum(o, axis=0, keepdims=True)[None]
        pq_ref[...] = jnp.sum(o * o, axis=0, keepdims=True)[None]

    return pl.pallas_call(
        body,
        grid=(NBLK,),
        in_specs=[
            pl.BlockSpec((n_chunks, BR, 128), lambda i: (0, i, 0)),
            pl.BlockSpec((BR, 1), lambda i: (i, 0)),
            pl.BlockSpec((1, d_out), lambda i: (0, 0)),
        ],
        out_specs=[
            pl.BlockSpec((BR, d_out), lambda i: (i, 0)),
            pl.BlockSpec((1, 1, d_out), lambda i: (i, 0, 0)),
            pl.BlockSpec((1, 1, d_out), lambda i: (i, 0, 0)),
        ],
        out_shape=[
            jax.ShapeDtypeStruct((N, d_out), jnp.float32),
            jax.ShapeDtypeStruct((NBLK, 1, d_out), jnp.float32),
            jax.ShapeDtypeStruct((NBLK, 1, d_out), jnp.float32),
        ],
    )(s, dis, b)


def _tc_pool(out2, ac2, batch):
    def body(o_ref, ac_ref, bt_ref, p_ref):
        i = pl.program_id(0)
        h = o_ref[...] * ac_ref[0:1] + ac_ref[1:2]
        h = _mish(h)
        bt = bt_ref[...]
        rows = [
            jnp.max(jnp.where(bt == g, h, -jnp.inf), axis=0, keepdims=True)
            for g in range(G)
        ]
        bm = jnp.concatenate(rows, axis=0)

        @pl.when(i == 0)
        def _():
            p_ref[...] = bm

        @pl.when(i > 0)
        def _():
            p_ref[...] = jnp.maximum(p_ref[...], bm)

    return pl.pallas_call(
        body,
        grid=(NBLK,),
        in_specs=[
            pl.BlockSpec((BR, D_OUT), lambda i: (i, 0)),
            pl.BlockSpec((2, D_OUT), lambda i: (0, 0)),
            pl.BlockSpec((BR, 1), lambda i: (i, 0)),
        ],
        out_specs=pl.BlockSpec((G, D_OUT), lambda i: (0, 0)),
        out_shape=jax.ShapeDtypeStruct((G, D_OUT), jnp.float32),
    )(out2, ac2, batch)


def _tc_decoder(p1, p2, Wm, bm, Wc, bc):
    def body(p1_ref, p2_ref, wm_ref, bm_ref, wc_ref, bc_ref, o_ref):
        gg = p1_ref[...] * p2_ref[...]
        t = jnp.dot(gg, wm_ref[...], preferred_element_type=jnp.float32)
        t = _mish(t + bm_ref[...])
        o = jnp.dot(t, wc_ref[...], preferred_element_type=jnp.float32)
        o_ref[...] = o + bc_ref[...]

    return pl.pallas_call(
        body,
        out_shape=jax.ShapeDtypeStruct((G, bc.shape[1]), jnp.float32),
    )(p1, p2, Wm, bm, Wc, bc)


def _encode_pool(x, edge_index, batch, W1, b1, g1, be1, W2, b2, g2, be2):
    src = edge_index[0]
    dst = edge_index[1]
    src_p = jnp.concatenate([src, jnp.zeros((E_PAD - E,), jnp.int32)])
    dst_p = jnp.concatenate([dst, jnp.full((E_PAD - E,), N, jnp.int32)])

    degf = _sc_degree(dst_p).reshape(2, N_ACC, 128)
    f1, dis = _tc_prep(x, degf, W1)
    s1 = _sc_prop6(f1.reshape(6 * N, 128), src_p, dst_p)
    s1 = s1.reshape(6, N_ACC, 128)
    out1, ps1, pq1 = _tc_recon(s1, dis, b1.reshape(1, D_H), D_H, 6)
    ac1 = _tc_stats(ps1, pq1, g1.reshape(1, D_H), be1.reshape(1, D_H))
    f2 = _tc_mid(out1, ac1, dis, W2)
    s2 = _sc_prop3(f2.reshape(3 * N, 128), src_p, dst_p)
    s2 = s2.reshape(3, N_ACC, 128)
    out2, ps2, pq2 = _tc_recon(s2, dis, b2.reshape(1, D_OUT), D_OUT, 3)
    ac2 = _tc_stats(ps2, pq2, g2.reshape(1, D_OUT), be2.reshape(1, D_OUT))
    return _tc_pool(out2, ac2, batch.reshape(N, 1))


def kernel(protein1_feat, protein1_edge_index, protein1_nodes_batch,
           protein2_feat, protein2_edge_index, protein2_nodes_batch,
           W1, b1, g1, be1, W2, b2, g2, be2, Wm, bm, Wc, bc):
    p1 = _encode_pool(protein1_feat, protein1_edge_index, protein1_nodes_batch,
                      W1, b1, g1, be1, W2, b2, g2, be2)
    p2 = _encode_pool(protein2_feat, protein2_edge_index, protein2_nodes_batch,
                      W1, b1, g1, be1, W2, b2, g2, be2)
    return _tc_decoder(p1, p2, Wm, bm.reshape(1, -1), Wc,
                       bc.reshape(1, -1))


# speedup vs baseline: 2.7699x; 1.1620x over previous
import functools

import jax
import jax.numpy as jnp
from jax import lax
from jax.experimental import pallas as pl
from jax.experimental.pallas import tpu as pltpu
from jax.experimental.pallas import tpu_sc as plsc

N = 10000
E = 160000
D_IN = 256
D_H = 720
D_OUT = 360
G = 8
EPS = 1e-5

NC = 2
NS = 16
NW = NC * NS
EB = 128
NB = 40
EW = EB * NB
E_PAD = EW * NW
N_ACC = 10240
RPT = N_ACC // NS

BR = 400
NBLK = N // BR

_mesh = plsc.VectorSubcoreMesh(core_axis_name="c", subcore_axis_name="s")


@functools.partial(
    pl.kernel,
    out_type=jax.ShapeDtypeStruct((2 * N_ACC, 128), jnp.float32),
    mesh=_mesh,
    scratch_types=[
        pltpu.VMEM((EB,), jnp.int32),
        pltpu.VMEM((EB, 128), jnp.float32),
        pltpu.VMEM((64, 128), jnp.float32),
        pltpu.VMEM_SHARED((N_ACC, 128), jnp.float32),
    ],
)
def _sc_degree(dst_hbm, out_hbm, idx_d, ones, zbuf, acc):
    c = lax.axis_index("c")
    s = lax.axis_index("s")
    w = s * NC + c
    o16 = jnp.ones((16,), jnp.float32)
    z16 = jnp.zeros((16,), jnp.float32)

    @pl.loop(0, EB)
    def _fill_ones(i):
        for j in range(8):
            ones[i, pl.ds(j * 16, 16)] = o16

    @pl.loop(0, 64)
    def _fill_zero(i):
        for j in range(8):
            zbuf[i, pl.ds(j * 16, 16)] = z16

    @pl.loop(0, RPT // 64)
    def _zero_acc(k):
        pltpu.sync_copy(zbuf, acc.at[pl.ds(s * RPT + k * 64, 64)])

    plsc.subcore_barrier()

    @pl.loop(0, NB)
    def _edges(b):
        pltpu.sync_copy(dst_hbm.at[pl.ds(w * EW + b * EB, EB)], idx_d)
        pltpu.sync_copy(ones, acc.at[idx_d], add=True)

    plsc.subcore_barrier()
    pltpu.sync_copy(acc.at[pl.ds(s * RPT, RPT)],
                    out_hbm.at[pl.ds(c * N_ACC + s * RPT, RPT)])


NBT = E_PAD // NS // EB
NSB = NBT // 2
EWT = NBT * EB


def _make_propagate(n_chunks):
    rounds = (n_chunks + NC - 1) // NC

    @functools.partial(
        pl.kernel,
        out_type=jax.ShapeDtypeStruct((n_chunks * N_ACC, 128), jnp.float32),
        mesh=_mesh,
        scratch_types=[
            pltpu.VMEM((EB,), jnp.int32),
            pltpu.VMEM((EB,), jnp.int32),
            pltpu.VMEM((EB,), jnp.int32),
            pltpu.VMEM((EB,), jnp.int32),
            pltpu.VMEM((EB, 128), jnp.float32),
            pltpu.VMEM((EB, 128), jnp.float32),
            pltpu.VMEM((16, 128), jnp.float32),
            pltpu.VMEM_SHARED((N_ACC, 128), jnp.float32),
            pltpu.SemaphoreType.DMA,
            pltpu.SemaphoreType.DMA,
            pltpu.SemaphoreType.DMA,
            pltpu.SemaphoreType.DMA,
        ],
    )
    def _prop(f_hbm, src_hbm, dst_hbm, out_hbm, src_a, src_b, dst_a, dst_b,
              rows_a, rows_b, zbuf, acc, gs_a, gs_b, ss_a, ss_b):
        c = lax.axis_index("c")
        s = lax.axis_index("s")
        z16 = jnp.zeros((16,), jnp.float32)

        @pl.loop(0, 16)
        def _fill_zero(i):
            for j in range(8):
                zbuf[i, pl.ds(j * 16, 16)] = z16

        bufs = ((src_a, dst_a, rows_a, gs_a, ss_a),
                (src_b, dst_b, rows_b, gs_b, ss_b))

        for r in range(rounds):
            chunk = r * NC + c

            @pl.when(chunk < n_chunks)
            def _round():
                @pl.loop(0, RPT // 16)
                def _zero_acc(k):
                    pltpu.sync_copy(zbuf, acc.at[pl.ds(s * RPT + k * 16, 16)])

                plsc.subcore_barrier()

                src_base = (chunk * NS + s) * EWT
                dst_base = s * EWT

                def fire_g(p, b):
                    sb, db, rw, gs, _ = bufs[p]
                    pltpu.sync_copy(src_hbm.at[pl.ds(src_base + b * EB, EB)],
                                    sb)
                    pltpu.sync_copy(dst_hbm.at[pl.ds(dst_base + b * EB, EB)],
                                    db)
                    pltpu.async_copy(f_hbm.at[sb], rw, gs)

                def wait_g(p):
                    sb, _, rw, gs, _ = bufs[p]
                    pltpu.make_async_copy(f_hbm.at[sb], rw, gs).wait()

                def fire_s(p):
                    _, db, rw, _, ss = bufs[p]
                    pltpu.async_copy(rw, acc.at[db], ss, add=True)

                def wait_s(p):
                    _, db, rw, _, ss = bufs[p]
                    pltpu.make_async_copy(rw, acc.at[db], ss).wait()

                fire_g(0, 0)
                wait_g(0)
                fire_g(1, 1)
                fire_s(0)

                @pl.loop(0, NBT // 2 - 1)
                def _pipe(u):
                    wait_g(1)
                    wait_s(0)
                    fire_g(0, 2 * u + 2)
                    fire_s(1)
                    wait_g(0)
                    wait_s(1)
                    fire_g(1, 2 * u + 3)
                    fire_s(0)

                wait_g(1)
                wait_s(0)
                fire_s(1)
                wait_s(1)

                plsc.subcore_barrier()
                pltpu.sync_copy(acc.at[pl.ds(s * RPT, RPT)],
                                out_hbm.at[pl.ds(chunk * N_ACC + s * RPT,
                                                 RPT)])

    return _prop


_sc_prop6 = _make_propagate(6)
_sc_prop3 = _make_propagate(3)


def _mish(x):
    return x * jnp.tanh(jax.nn.softplus(x))


def _tc_prep(x, degf, W1):
    def body(x_ref, d_ref, w_ref, f_ref, dis_ref):
        deg = d_ref[0, :, 0:1] + d_ref[1, :, 0:1]
        dis = jnp.where(deg > 0, lax.rsqrt(deg), 0.0)
        h = jnp.dot(x_ref[...], w_ref[...],
                    preferred_element_type=jnp.float32)
        f = h * dis
        zp = jnp.concatenate([f, jnp.zeros((BR, 48), jnp.float32)], axis=1)
        for k in range(6):
            f_ref[k] = zp[:, 128 * k:128 * (k + 1)]
        dis_ref[...] = dis

    return pl.pallas_call(
        body,
        grid=(NBLK,),
        in_specs=[
            pl.BlockSpec((BR, D_IN), lambda i: (i, 0)),
            pl.BlockSpec((2, BR, 128), lambda i: (0, i, 0)),
            pl.BlockSpec((D_IN, D_H), lambda i: (0, 0)),
        ],
        out_specs=[
            pl.BlockSpec((6, BR, 128), lambda i: (0, i, 0)),
            pl.BlockSpec((BR, 1), lambda i: (i, 0)),
        ],
        out_shape=[
            jax.ShapeDtypeStruct((6, N, 128), jnp.float32),
            jax.ShapeDtypeStruct((N, 1), jnp.float32),
        ],
    )(x, degf, W1)


def _tc_stats(ps, pq, g, be):
    d = ps.shape[-1]

    def body(ps_ref, pq_ref, g_ref, be_ref, ac_ref):
        mean = jnp.sum(ps_ref[...], axis=0) * (1.0 / N)
        ex2 = jnp.sum(pq_ref[...], axis=0) * (1.0 / N)
        var = ex2 - mean * mean
        a = g_ref[...] * lax.rsqrt(var + EPS)
        c = be_ref[...] - mean * a
        ac_ref[...] = jnp.concatenate([a, c], axis=0)

    return pl.pallas_call(
        body,
        out_shape=jax.ShapeDtypeStruct((2, d), jnp.float32),
    )(ps, pq, g, be)


def _tc_mid(out1, ac1, dis, W2):
    def body(o_ref, ac_ref, dis_ref, w_ref, f_ref):
        h = o_ref[...] * ac_ref[0:1] + ac_ref[1:2]
        h = _mish(h)
        z = jnp.dot(h, w_ref[...], preferred_element_type=jnp.float32)
        zd = z * dis_ref[...]
        zp = jnp.concatenate([zd, jnp.zeros((BR, 24), jnp.float32)], axis=1)
        f_ref[0] = zp[:, :128]
        f_ref[1] = zp[:, 128:256]
        f_ref[2] = zp[:, 256:]

    return pl.pallas_call(
        body,
        grid=(NBLK,),
        in_specs=[
            pl.BlockSpec((BR, D_H), lambda i: (i, 0)),
            pl.BlockSpec((2, D_H), lambda i: (0, 0)),
            pl.BlockSpec((BR, 1), lambda i: (i, 0)),
            pl.BlockSpec((D_H, D_OUT), lambda i: (0, 0)),
        ],
        out_specs=pl.BlockSpec((3, BR, 128), lambda i: (0, i, 0)),
        out_shape=jax.ShapeDtypeStruct((3, N, 128), jnp.float32),
    )(out1, ac1, dis, W2)


def _tc_recon(s, dis, b, d_out, n_chunks):
    def body(s_ref, dis_ref, b_ref, o_ref, ps_ref, pq_ref):
        t = jnp.concatenate([s_ref[k] for k in range(n_chunks)],
                            axis=1)[:, :d_out]
        o = t * dis_ref[...] + b_ref[...]
        o_ref[...] = o
        ps_ref[...] = jnp.sum(o, axis=0, keepdims=True)[None]
        pq_ref[...] = jnp.sum(o * o, axis=0, keepdims=True)[None]

    return pl.pallas_call(
        body,
        grid=(NBLK,),
        in_specs=[
            pl.BlockSpec((n_chunks, BR, 128), lambda i: (0, i, 0)),
            pl.BlockSpec((BR, 1), lambda i: (i, 0)),
            pl.BlockSpec((1, d_out), lambda i: (0, 0)),
        ],
        out_specs=[
            pl.BlockSpec((BR, d_out), lambda i: (i, 0)),
            pl.BlockSpec((1, 1, d_out), lambda i: (i, 0, 0)),
            pl.BlockSpec((1, 1, d_out), lambda i: (i, 0, 0)),
        ],
        out_shape=[
            jax.ShapeDtypeStruct((N, d_out), jnp.float32),
            jax.ShapeDtypeStruct((NBLK, 1, d_out), jnp.float32),
            jax.ShapeDtypeStruct((NBLK, 1, d_out), jnp.float32),
        ],
    )(s, dis, b)


def _tc_pool(out2, ac2, batch):
    def body(o_ref, ac_ref, bt_ref, p_ref):
        i = pl.program_id(0)
        h = o_ref[...] * ac_ref[0:1] + ac_ref[1:2]
        h = _mish(h)
        bt = bt_ref[...]
        rows = [
            jnp.max(jnp.where(bt == g, h, -jnp.inf), axis=0, keepdims=True)
            for g in range(G)
        ]
        bm = jnp.concatenate(rows, axis=0)

        @pl.when(i == 0)
        def _():
            p_ref[...] = bm

        @pl.when(i > 0)
        def _():
            p_ref[...] = jnp.maximum(p_ref[...], bm)

    return pl.pallas_call(
        body,
        grid=(NBLK,),
        in_specs=[
            pl.BlockSpec((BR, D_OUT), lambda i: (i, 0)),
            pl.BlockSpec((2, D_OUT), lambda i: (0, 0)),
            pl.BlockSpec((BR, 1), lambda i: (i, 0)),
        ],
        out_specs=pl.BlockSpec((G, D_OUT), lambda i: (0, 0)),
        out_shape=jax.ShapeDtypeStruct((G, D_OUT), jnp.float32),
    )(out2, ac2, batch)


def _tc_decoder(p1, p2, Wm, bm, Wc, bc):
    def body(p1_ref, p2_ref, wm_ref, bm_ref, wc_ref, bc_ref, o_ref):
        gg = p1_ref[...] * p2_ref[...]
        t = jnp.dot(gg, wm_ref[...], preferred_element_type=jnp.float32)
        t = _mish(t + bm_ref[...])
        o = jnp.dot(t, wc_ref[...], preferred_element_type=jnp.float32)
        o_ref[...] = o + bc_ref[...]

    return pl.pallas_call(
        body,
        out_shape=jax.ShapeDtypeStruct((G, bc.shape[1]), jnp.float32),
    )(p1, p2, Wm, bm, Wc, bc)


def _encode_pool(x, edge_index, batch, W1, b1, g1, be1, W2, b2, g2, be2):
    src = edge_index[0]
    dst = edge_index[1]
    src_p = jnp.concatenate([src, jnp.zeros((E_PAD - E,), jnp.int32)])
    dst_p = jnp.concatenate([dst, jnp.full((E_PAD - E,), N, jnp.int32)])
    src6 = jnp.concatenate([src_p + k * N for k in range(6)])
    src3 = jnp.concatenate([src_p + k * N for k in range(3)])

    degf = _sc_degree(dst_p).reshape(2, N_ACC, 128)
    f1, dis = _tc_prep(x, degf, W1)
    s1 = _sc_prop6(f1.reshape(6 * N, 128), src6, dst_p)
    s1 = s1.reshape(6, N_ACC, 128)
    out1, ps1, pq1 = _tc_recon(s1, dis, b1.reshape(1, D_H), D_H, 6)
    ac1 = _tc_stats(ps1, pq1, g1.reshape(1, D_H), be1.reshape(1, D_H))
    f2 = _tc_mid(out1, ac1, dis, W2)
    s2 = _sc_prop3(f2.reshape(3 * N, 128), src3, dst_p)
    s2 = s2.reshape(3, N_ACC, 128)
    out2, ps2, pq2 = _tc_recon(s2, dis, b2.reshape(1, D_OUT), D_OUT, 3)
    ac2 = _tc_stats(ps2, pq2, g2.reshape(1, D_OUT), be2.reshape(1, D_OUT))
    return _tc_pool(out2, ac2, batch.reshape(N, 1))


def kernel(protein1_feat, protein1_edge_index, protein1_nodes_batch,
           protein2_feat, protein2_edge_index, protein2_nodes_batch,
           W1, b1, g1, be1, W2, b2, g2, be2, Wm, bm, Wc, bc):
    p1 = _encode_pool(protein1_feat, protein1_edge_index, protein1_nodes_batch,
                      W1, b1, g1, be1, W2, b2, g2, be2)
    p2 = _encode_pool(protein2_feat, protein2_edge_index, protein2_nodes_batch,
                      W1, b1, g1, be1, W2, b2, g2, be2)
    return _tc_decoder(p1, p2, Wm, bm.reshape(1, -1), Wc,
                       bc.reshape(1, -1))


# speedup vs baseline: 3.1101x; 1.1228x over previous
import functools

import jax
import jax.numpy as jnp
from jax import lax
from jax.experimental import pallas as pl
from jax.experimental.pallas import tpu as pltpu
from jax.experimental.pallas import tpu_sc as plsc

N = 10000
E = 160000
D_IN = 256
D_H = 720
D_OUT = 360
G = 8
EPS = 1e-5

NC = 2
NS = 16
NW = NC * NS
EB = 128
NB = 40
EW = EB * NB
E_PAD = EW * NW
N_ACC = 10240
RPT = N_ACC // NS

BR = 400
NBLK = N // BR

_mesh = plsc.VectorSubcoreMesh(core_axis_name="c", subcore_axis_name="s")


@functools.partial(
    pl.kernel,
    out_type=jax.ShapeDtypeStruct((2 * N_ACC, 128), jnp.float32),
    mesh=_mesh,
    scratch_types=[
        pltpu.VMEM((EB,), jnp.int32),
        pltpu.VMEM((EB, 128), jnp.float32),
        pltpu.VMEM((64, 128), jnp.float32),
        pltpu.VMEM_SHARED((N_ACC, 128), jnp.float32),
    ],
)
def _sc_degree(dst_hbm, out_hbm, idx_d, ones, zbuf, acc):
    c = lax.axis_index("c")
    s = lax.axis_index("s")
    w = s * NC + c
    o16 = jnp.ones((16,), jnp.float32)
    z16 = jnp.zeros((16,), jnp.float32)

    @pl.loop(0, EB)
    def _fill_ones(i):
        for j in range(8):
            ones[i, pl.ds(j * 16, 16)] = o16

    @pl.loop(0, 64)
    def _fill_zero(i):
        for j in range(8):
            zbuf[i, pl.ds(j * 16, 16)] = z16

    @pl.loop(0, RPT // 64)
    def _zero_acc(k):
        pltpu.sync_copy(zbuf, acc.at[pl.ds(s * RPT + k * 64, 64)])

    plsc.subcore_barrier()

    @pl.loop(0, NB)
    def _edges(b):
        pltpu.sync_copy(dst_hbm.at[pl.ds(w * EW + b * EB, EB)], idx_d)
        pltpu.sync_copy(ones, acc.at[idx_d], add=True)

    plsc.subcore_barrier()
    pltpu.sync_copy(acc.at[pl.ds(s * RPT, RPT)],
                    out_hbm.at[pl.ds(c * N_ACC + s * RPT, RPT)])


NBT = E_PAD // NS // EB
EWT = NBT * EB
GB = 8


def _make_propagate(n_chunks):
    rounds = (n_chunks + NC - 1) // NC

    @functools.partial(
        pl.kernel,
        out_type=jax.ShapeDtypeStruct((n_chunks * N_ACC, 128), jnp.float32),
        mesh=_mesh,
        scratch_types=[
            pltpu.VMEM((GB, EB), jnp.int32),
            pltpu.VMEM((GB, EB), jnp.int32),
            pltpu.VMEM((EB, 128), jnp.float32),
            pltpu.VMEM((EB, 128), jnp.float32),
            pltpu.VMEM((16, 128), jnp.float32),
            pltpu.VMEM_SHARED((N_ACC, 128), jnp.float32),
            pltpu.SemaphoreType.DMA,
            pltpu.SemaphoreType.DMA,
            pltpu.SemaphoreType.DMA,
            pltpu.SemaphoreType.DMA,
        ],
    )
    def _prop(f_hbm, src_hbm, dst_hbm, out_hbm, src_blk, dst_blk,
              rows_a, rows_b, zbuf, acc, gs_a, gs_b, ss_a, ss_b):
        c = lax.axis_index("c")
        s = lax.axis_index("s")
        z16 = jnp.zeros((16,), jnp.float32)

        @pl.loop(0, 16)
        def _fill_zero(i):
            for j in range(8):
                zbuf[i, pl.ds(j * 16, 16)] = z16

        bufs = ((rows_a, gs_a, ss_a), (rows_b, gs_b, ss_b))

        for r in range(rounds):
            chunk = r * NC + c

            @pl.when(chunk < n_chunks)
            def _round():
                @pl.loop(0, RPT // 16)
                def _zero_acc(k):
                    pltpu.sync_copy(zbuf, acc.at[pl.ds(s * RPT + k * 16, 16)])

                plsc.subcore_barrier()

                src_row0 = (chunk * NS + s) * NBT
                dst_row0 = s * NBT

                def fire_g(p, j):
                    rw, gs, _ = bufs[p]
                    pltpu.async_copy(f_hbm.at[src_blk.at[j]], rw, gs)

                def wait_g(p):
                    rw, gs, _ = bufs[p]
                    pltpu.make_async_copy(f_hbm.at[src_blk.at[0]], rw,
                                          gs).wait()

                def fire_s(p, j):
                    rw, _, ss = bufs[p]
                    pltpu.async_copy(rw, acc.at[dst_blk.at[j]], ss, add=True)

                def wait_s(p):
                    rw, _, ss = bufs[p]
                    pltpu.make_async_copy(rw, acc.at[dst_blk.at[0]],
                                          ss).wait()

                @pl.loop(0, NBT // GB)
                def _grp(g):
                    pltpu.sync_copy(src_hbm.at[pl.ds(src_row0 + g * GB, GB)],
                                    src_blk)
                    pltpu.sync_copy(dst_hbm.at[pl.ds(dst_row0 + g * GB, GB)],
                                    dst_blk)
                    fire_g(0, 0)
                    wait_g(0)
                    fire_g(1, 1)
                    fire_s(0, 0)
                    for j in range(1, GB // 2):
                        wait_g(1)
                        wait_s(0)
                        fire_g(0, 2 * j)
                        fire_s(1, 2 * j - 1)
                        wait_g(0)
                        wait_s(1)
                        fire_g(1, 2 * j + 1)
                        fire_s(0, 2 * j)
                    wait_g(1)
                    wait_s(0)
                    fire_s(1, GB - 1)
                    wait_s(1)

                plsc.subcore_barrier()
                pltpu.sync_copy(acc.at[pl.ds(s * RPT, RPT)],
                                out_hbm.at[pl.ds(chunk * N_ACC + s * RPT,
                                                 RPT)])

    return _prop


_sc_prop6 = _make_propagate(6)
_sc_prop3 = _make_propagate(3)


def _mish(x):
    return x * jnp.tanh(jax.nn.softplus(x))


def _tc_prep(x, degf, W1):
    def body(x_ref, d_ref, w_ref, f_ref, dis_ref):
        deg = d_ref[0, :, 0:1] + d_ref[1, :, 0:1]
        dis = jnp.where(deg > 0, lax.rsqrt(deg), 0.0)
        h = jnp.dot(x_ref[...], w_ref[...],
                    preferred_element_type=jnp.float32)
        f = h * dis
        zp = jnp.concatenate([f, jnp.zeros((BR, 48), jnp.float32)], axis=1)
        for k in range(6):
            f_ref[k] = zp[:, 128 * k:128 * (k + 1)]
        dis_ref[...] = dis

    return pl.pallas_call(
        body,
        grid=(NBLK,),
        in_specs=[
            pl.BlockSpec((BR, D_IN), lambda i: (i, 0)),
            pl.BlockSpec((2, BR, 128), lambda i: (0, i, 0)),
            pl.BlockSpec((D_IN, D_H), lambda i: (0, 0)),
        ],
        out_specs=[
            pl.BlockSpec((6, BR, 128), lambda i: (0, i, 0)),
            pl.BlockSpec((BR, 1), lambda i: (i, 0)),
        ],
        out_shape=[
            jax.ShapeDtypeStruct((6, N, 128), jnp.float32),
            jax.ShapeDtypeStruct((N, 1), jnp.float32),
        ],
    )(x, degf, W1)


def _tc_stats(ps, pq, g, be):
    d = ps.shape[-1]

    def body(ps_ref, pq_ref, g_ref, be_ref, ac_ref):
        mean = jnp.sum(ps_ref[...], axis=0) * (1.0 / N)
        ex2 = jnp.sum(pq_ref[...], axis=0) * (1.0 / N)
        var = ex2 - mean * mean
        a = g_ref[...] * lax.rsqrt(var + EPS)
        c = be_ref[...] - mean * a
        ac_ref[...] = jnp.concatenate([a, c], axis=0)

    return pl.pallas_call(
        body,
        out_shape=jax.ShapeDtypeStruct((2, d), jnp.float32),
    )(ps, pq, g, be)


def _tc_mid(out1, ac1, dis, W2):
    def body(o_ref, ac_ref, dis_ref, w_ref, f_ref):
        h = o_ref[...] * ac_ref[0:1] + ac_ref[1:2]
        h = _mish(h)
        z = jnp.dot(h, w_ref[...], preferred_element_type=jnp.float32)
        zd = z * dis_ref[...]
        zp = jnp.concatenate([zd, jnp.zeros((BR, 24), jnp.float32)], axis=1)
        f_ref[0] = zp[:, :128]
        f_ref[1] = zp[:, 128:256]
        f_ref[2] = zp[:, 256:]

    return pl.pallas_call(
        body,
        grid=(NBLK,),
        in_specs=[
            pl.BlockSpec((BR, D_H), lambda i: (i, 0)),
            pl.BlockSpec((2, D_H), lambda i: (0, 0)),
            pl.BlockSpec((BR, 1), lambda i: (i, 0)),
            pl.BlockSpec((D_H, D_OUT), lambda i: (0, 0)),
        ],
        out_specs=pl.BlockSpec((3, BR, 128), lambda i: (0, i, 0)),
        out_shape=jax.ShapeDtypeStruct((3, N, 128), jnp.float32),
    )(out1, ac1, dis, W2)


def _tc_recon(s, dis, b, d_out, n_chunks):
    def body(s_ref, dis_ref, b_ref, o_ref, ps_ref, pq_ref):
        t = jnp.concatenate([s_ref[k] for k in range(n_chunks)],
                            axis=1)[:, :d_out]
        o = t * dis_ref[...] + b_ref[...]
        o_ref[...] = o
        ps_ref[...] = jnp.sum(o, axis=0, keepdims=True)[None]
        pq_ref[...] = jnp.sum(o * o, axis=0, keepdims=True)[None]

    return pl.pallas_call(
        body,
        grid=(NBLK,),
        in_specs=[
            pl.BlockSpec((n_chunks, BR, 128), lambda i: (0, i, 0)),
            pl.BlockSpec((BR, 1), lambda i: (i, 0)),
            pl.BlockSpec((1, d_out), lambda i: (0, 0)),
        ],
        out_specs=[
            pl.BlockSpec((BR, d_out), lambda i: (i, 0)),
            pl.BlockSpec((1, 1, d_out), lambda i: (i, 0, 0)),
            pl.BlockSpec((1, 1, d_out), lambda i: (i, 0, 0)),
        ],
        out_shape=[
            jax.ShapeDtypeStruct((N, d_out), jnp.float32),
            jax.ShapeDtypeStruct((NBLK, 1, d_out), jnp.float32),
            jax.ShapeDtypeStruct((NBLK, 1, d_out), jnp.float32),
        ],
    )(s, dis, b)


def _tc_pool(out2, ac2, batch):
    def body(o_ref, ac_ref, bt_ref, p_ref):
        i = pl.program_id(0)
        h = o_ref[...] * ac_ref[0:1] + ac_ref[1:2]
        h = _mish(h)
        bt = bt_ref[...]
        rows = [
            jnp.max(jnp.where(bt == g, h, -jnp.inf), axis=0, keepdims=True)
            for g in range(G)
        ]
        bm = jnp.concatenate(rows, axis=0)

        @pl.when(i == 0)
        def _():
            p_ref[...] = bm

        @pl.when(i > 0)
        def _():
            p_ref[...] = jnp.maximum(p_ref[...], bm)

    return pl.pallas_call(
        body,
        grid=(NBLK,),
        in_specs=[
            pl.BlockSpec((BR, D_OUT), lambda i: (i, 0)),
            pl.BlockSpec((2, D_OUT), lambda i: (0, 0)),
            pl.BlockSpec((BR, 1), lambda i: (i, 0)),
        ],
        out_specs=pl.BlockSpec((G, D_OUT), lambda i: (0, 0)),
        out_shape=jax.ShapeDtypeStruct((G, D_OUT), jnp.float32),
    )(out2, ac2, batch)


def _tc_decoder(p1, p2, Wm, bm, Wc, bc):
    def body(p1_ref, p2_ref, wm_ref, bm_ref, wc_ref, bc_ref, o_ref):
        gg = p1_ref[...] * p2_ref[...]
        t = jnp.dot(gg, wm_ref[...], preferred_element_type=jnp.float32)
        t = _mish(t + bm_ref[...])
        o = jnp.dot(t, wc_ref[...], preferred_element_type=jnp.float32)
        o_ref[...] = o + bc_ref[...]

    return pl.pallas_call(
        body,
        out_shape=jax.ShapeDtypeStruct((G, bc.shape[1]), jnp.float32),
    )(p1, p2, Wm, bm, Wc, bc)


def _encode_pool(x, edge_index, batch, W1, b1, g1, be1, W2, b2, g2, be2):
    src = edge_index[0]
    dst = edge_index[1]
    src_p = jnp.concatenate([src, jnp.zeros((E_PAD - E,), jnp.int32)])
    dst_p = jnp.concatenate([dst, jnp.full((E_PAD - E,), N, jnp.int32)])
    src6 = jnp.concatenate([src_p + k * N for k in range(6)]).reshape(-1, EB)
    src3 = jnp.concatenate([src_p + k * N for k in range(3)]).reshape(-1, EB)
    dst_2d = dst_p.reshape(-1, EB)

    degf = _sc_degree(dst_p).reshape(2, N_ACC, 128)
    f1, dis = _tc_prep(x, degf, W1)
    s1 = _sc_prop6(f1.reshape(6 * N, 128), src6, dst_2d)
    s1 = s1.reshape(6, N_ACC, 128)
    out1, ps1, pq1 = _tc_recon(s1, dis, b1.reshape(1, D_H), D_H, 6)
    ac1 = _tc_stats(ps1, pq1, g1.reshape(1, D_H), be1.reshape(1, D_H))
    f2 = _tc_mid(out1, ac1, dis, W2)
    s2 = _sc_prop3(f2.reshape(3 * N, 128), src3, dst_2d)
    s2 = s2.reshape(3, N_ACC, 128)
    out2, ps2, pq2 = _tc_recon(s2, dis, b2.reshape(1, D_OUT), D_OUT, 3)
    ac2 = _tc_stats(ps2, pq2, g2.reshape(1, D_OUT), be2.reshape(1, D_OUT))
    return _tc_pool(out2, ac2, batch.reshape(N, 1))


def kernel(protein1_feat, protein1_edge_index, protein1_nodes_batch,
           protein2_feat, protein2_edge_index, protein2_nodes_batch,
           W1, b1, g1, be1, W2, b2, g2, be2, Wm, bm, Wc, bc):
    p1 = _encode_pool(protein1_feat, protein1_edge_index, protein1_nodes_batch,
                      W1, b1, g1, be1, W2, b2, g2, be2)
    p2 = _encode_pool(protein2_feat, protein2_edge_index, protein2_nodes_batch,
                      W1, b1, g1, be1, W2, b2, g2, be2)
    return _tc_decoder(p1, p2, Wm, bm.reshape(1, -1), Wc,
                       bc.reshape(1, -1))
